# baseline (speedup 1.0000x reference)
import functools

import jax
import jax.numpy as jnp
from jax import lax
from jax.experimental import pallas as pl
from jax.experimental.pallas import tpu as pltpu
from jax.experimental.pallas import tpu_sc as plsc

N_COMPONENT = 64
H_HOP = 2
LAYER_L = 2
DEG = 32

N = 10000
KDIM = 64

NW = 32
NODES_PER_W = 320
NPAD = NW * NODES_PER_W
CHUNK = 4
IDXS = CHUNK * DEG
NCHUNKS = NODES_PER_W // CHUNK
NSUB = KDIM // 16


def _make_hop():
    def body(adj_hbm, t_hbm, out_hbm, idx0_v, idx1_v, rows0_v, rows1_v,
             out_v, sem0, sem1):
        wid = lax.axis_index("s") * 2 + lax.axis_index("c")
        base = wid * NODES_PER_W
        idxb = (idx0_v, idx1_v)
        rows = (rows0_v, rows1_v)
        sems = (sem0, sem1)

        def start(chunk, buf):
            pltpu.sync_copy(
                adj_hbm.at[pl.ds((base + chunk * CHUNK) * DEG, IDXS)],
                idxb[buf])
            pltpu.async_copy(t_hbm.at[idxb[buf]], rows[buf], sems[buf])

        start(0, 0)

        @pl.loop(0, NCHUNKS, step=2)
        def chunk_pair(g):
            for b in (0, 1):
                gg = g + b
                nxt = gg + 1

                @pl.when(nxt < NCHUNKS)
                def _():
                    start(nxt, 1 - b)

                pltpu.make_async_copy(
                    t_hbm.at[idxb[b]], rows[b], sems[b]).wait()
                for n in range(CHUNK):
                    for f in range(NSUB):
                        acc = rows[b][n * DEG, pl.ds(16 * f, 16)]
                        for j in range(1, DEG):
                            acc = acc + rows[b][n * DEG + j, pl.ds(16 * f, 16)]
                        out_v[gg * CHUNK + n, pl.ds(16 * f, 16)] = (
                            acc * (1.0 / DEG))

        pltpu.sync_copy(out_v, out_hbm.at[pl.ds(base, NODES_PER_W)])

    return pl.kernel(
        body,
        out_type=jax.ShapeDtypeStruct((NPAD, KDIM), jnp.float32),
        mesh=plsc.VectorSubcoreMesh(core_axis_name="c", subcore_axis_name="s"),
        scratch_types=[
            pltpu.VMEM((IDXS,), jnp.int32),
            pltpu.VMEM((IDXS,), jnp.int32),
            pltpu.VMEM((IDXS, KDIM), jnp.float32),
            pltpu.VMEM((IDXS, KDIM), jnp.float32),
            pltpu.VMEM((NODES_PER_W, KDIM), jnp.float32),
            pltpu.SemaphoreType.DMA,
            pltpu.SemaphoreType.DMA,
        ],
        compiler_params=pltpu.CompilerParams(use_tc_tiling_on_sc=False),
    )


_hop = _make_hop()


def _ew1_body(d_ref, o_ref):
    e = jnp.exp(d_ref[...])
    s = jnp.sum(e, axis=1, keepdims=True)
    o_ref[...] = e / jnp.where(s == 0.0, 1.0, s)


_EW_BLOCK = 1000


def _ew1(d):
    return pl.pallas_call(
        _ew1_body,
        grid=(N // _EW_BLOCK,),
        in_specs=[pl.BlockSpec((_EW_BLOCK, KDIM), lambda i: (i, 0))],
        out_specs=pl.BlockSpec((_EW_BLOCK, KDIM), lambda i: (i, 0)),
        out_shape=jax.ShapeDtypeStruct((N, KDIM), jnp.float32),
    )(d)


def _ew2_body(d_ref, o_ref):
    o_ref[...] = jnp.log(jnp.clip(d_ref[...], 1e-9, None))


def _ew2(d):
    return pl.pallas_call(
        _ew2_body,
        grid=(N // _EW_BLOCK,),
        in_specs=[pl.BlockSpec((_EW_BLOCK, KDIM), lambda i: (i, 0))],
        out_specs=pl.BlockSpec((_EW_BLOCK, KDIM), lambda i: (i, 0)),
        out_shape=jax.ShapeDtypeStruct((N, KDIM), jnp.float32),
    )(d)


def _fin_body(q1_ref, q2_ref, b_ref, o_ref):
    o_ref[...] = jnp.dot(
        q1_ref[...] + q2_ref[...],
        b_ref[...],
        precision=lax.Precision.HIGHEST,
        preferred_element_type=jnp.float32,
    )


_FIN_BLOCK = 1000


def _final(q1, q2, b2):
    return pl.pallas_call(
        _fin_body,
        grid=(N // _FIN_BLOCK,),
        in_specs=[
            pl.BlockSpec((_FIN_BLOCK, KDIM), lambda i: (i, 0)),
            pl.BlockSpec((_FIN_BLOCK, KDIM), lambda i: (i, 0)),
            pl.BlockSpec((KDIM, 128), lambda i: (0, 0)),
        ],
        out_specs=pl.BlockSpec((_FIN_BLOCK, 128), lambda i: (i, 0)),
        out_shape=jax.ShapeDtypeStruct((N, 128), jnp.float32),
    )(q1, q2, b2)


def kernel(x, adj_index):
    adj = adj_index.astype(jnp.int32)
    adj = jnp.pad(adj, ((0, NPAD - N), (0, 0)))
    adj_flat = adj.reshape(-1)

    U, S, Vt = jnp.linalg.svd(x, full_matrices=False)
    dis = U[:, :N_COMPONENT]
    s64 = S[:N_COMPONENT]
    base = Vt[:N_COMPONENT, :]

    qs = []
    for _ in range(LAYER_L):
        t = _ew1(dis)
        h1 = _hop(adj_flat, t)
        h2 = _hop(adj_flat, h1)
        a = _ew2(h2)
        q, _ = jnp.linalg.qr(a)
        dis = q
        qs.append(q)

    b2 = (0.5 * s64)[:, None] * base
    return _final(qs[0], qs[1], b2)


# speedup vs baseline: 1.0078x; 1.0078x over previous
import functools

import jax
import jax.numpy as jnp
from jax import lax
from jax.experimental import pallas as pl
from jax.experimental.pallas import tpu as pltpu
from jax.experimental.pallas import tpu_sc as plsc

N_COMPONENT = 64
H_HOP = 2
LAYER_L = 2
DEG = 32

N = 10000
KDIM = 64

NW = 32
NODES_PER_W = 320
NPAD = NW * NODES_PER_W
CHUNK = 4
IDXS = CHUNK * DEG
NCHUNKS = NODES_PER_W // CHUNK
NSUB = KDIM // 16


def _make_hop():
    NBUF = 4

    def body(adj_hbm, t_hbm, out_hbm, idx_v, rows_v, out_v, sems):
        wid = lax.axis_index("s") * 2 + lax.axis_index("c")
        base = wid * NODES_PER_W

        pltpu.sync_copy(adj_hbm.at[wid], idx_v)

        def start(chunk, buf):
            pltpu.async_copy(t_hbm.at[idx_v.at[chunk]],
                             rows_v[buf], sems[buf])

        for p in range(NBUF - 1):
            start(p, p)

        @pl.loop(0, NCHUNKS, step=NBUF)
        def chunk_quad(g):
            for b in range(NBUF):
                gg = g + b
                nxt = gg + (NBUF - 1)

                @pl.when(nxt < NCHUNKS)
                def _():
                    start(nxt, (b + NBUF - 1) % NBUF)

                pltpu.make_async_copy(
                    t_hbm.at[idx_v.at[0]], rows_v[b], sems[b]).wait()
                for n in range(CHUNK):
                    for f in range(NSUB):
                        acc = rows_v[b][n * DEG, pl.ds(16 * f, 16)]
                        for j in range(1, DEG):
                            acc = acc + rows_v[b][n * DEG + j,
                                                  pl.ds(16 * f, 16)]
                        out_v[gg * CHUNK + n, pl.ds(16 * f, 16)] = (
                            acc * (1.0 / DEG))

        pltpu.sync_copy(out_v, out_hbm.at[pl.ds(base, NODES_PER_W)])

    return pl.kernel(
        body,
        out_type=jax.ShapeDtypeStruct((NPAD, KDIM), jnp.float32),
        mesh=plsc.VectorSubcoreMesh(core_axis_name="c", subcore_axis_name="s"),
        scratch_types=[
            pltpu.VMEM((NCHUNKS, IDXS), jnp.int32),
            [pltpu.VMEM((IDXS, KDIM), jnp.float32) for _ in range(NBUF)],
            pltpu.VMEM((NODES_PER_W, KDIM), jnp.float32),
            [pltpu.SemaphoreType.DMA for _ in range(NBUF)],
        ],
        compiler_params=pltpu.CompilerParams(use_tc_tiling_on_sc=False),
    )


_hop = _make_hop()


def _ew1_body(d_ref, o_ref):
    e = jnp.exp(d_ref[...])
    s = jnp.sum(e, axis=1, keepdims=True)
    o_ref[...] = e / jnp.where(s == 0.0, 1.0, s)


_EW_BLOCK = 1000


def _ew1(d):
    return pl.pallas_call(
        _ew1_body,
        grid=(N // _EW_BLOCK,),
        in_specs=[pl.BlockSpec((_EW_BLOCK, KDIM), lambda i: (i, 0))],
        out_specs=pl.BlockSpec((_EW_BLOCK, KDIM), lambda i: (i, 0)),
        out_shape=jax.ShapeDtypeStruct((N, KDIM), jnp.float32),
    )(d)


def _ew2_body(d_ref, o_ref):
    o_ref[...] = jnp.log(jnp.clip(d_ref[...], 1e-9, None))


def _ew2(d):
    return pl.pallas_call(
        _ew2_body,
        grid=(N // _EW_BLOCK,),
        in_specs=[pl.BlockSpec((_EW_BLOCK, KDIM), lambda i: (i, 0))],
        out_specs=pl.BlockSpec((_EW_BLOCK, KDIM), lambda i: (i, 0)),
        out_shape=jax.ShapeDtypeStruct((N, KDIM), jnp.float32),
    )(d)


def _fin_body(q1_ref, q2_ref, b_ref, o_ref):
    o_ref[...] = jnp.dot(
        q1_ref[...] + q2_ref[...],
        b_ref[...],
        precision=lax.Precision.HIGHEST,
        preferred_element_type=jnp.float32,
    )


_FIN_BLOCK = 1000


def _final(q1, q2, b2):
    return pl.pallas_call(
        _fin_body,
        grid=(N // _FIN_BLOCK,),
        in_specs=[
            pl.BlockSpec((_FIN_BLOCK, KDIM), lambda i: (i, 0)),
            pl.BlockSpec((_FIN_BLOCK, KDIM), lambda i: (i, 0)),
            pl.BlockSpec((KDIM, 128), lambda i: (0, 0)),
        ],
        out_specs=pl.BlockSpec((_FIN_BLOCK, 128), lambda i: (i, 0)),
        out_shape=jax.ShapeDtypeStruct((N, 128), jnp.float32),
    )(q1, q2, b2)


def kernel(x, adj_index):
    adj = adj_index.astype(jnp.int32)
    adj = jnp.pad(adj, ((0, NPAD - N), (0, 0)))
    adj_flat = adj.reshape(NW, NCHUNKS, IDXS)

    U, S, Vt = jnp.linalg.svd(x, full_matrices=False)
    dis = U[:, :N_COMPONENT]
    s64 = S[:N_COMPONENT]
    base = Vt[:N_COMPONENT, :]

    qs = []
    for _ in range(LAYER_L):
        t = _ew1(dis)
        h1 = _hop(adj_flat, t)
        h2 = _hop(adj_flat, h1)
        a = _ew2(h2)
        q, _ = jnp.linalg.qr(a)
        dis = q
        qs.append(q)

    b2 = (0.5 * s64)[:, None] * base
    return _final(qs[0], qs[1], b2)


# speedup vs baseline: 1.1156x; 1.1069x over previous
import functools

import jax
import jax.numpy as jnp
from jax import lax
from jax.experimental import pallas as pl
from jax.experimental.pallas import tpu as pltpu
from jax.experimental.pallas import tpu_sc as plsc

N_COMPONENT = 64
H_HOP = 2
LAYER_L = 2
DEG = 32

N = 10000
KDIM = 64

NW = 32
NODES_PER_W = 320
NPAD = NW * NODES_PER_W
CHUNK = 4
IDXS = CHUNK * DEG
NCHUNKS = NODES_PER_W // CHUNK
NSUB = KDIM // 16


def _make_hop(table_rows: int):
    NBUF = 4
    STAGE = table_rows // 16

    def body(adj_hbm, t_hbm, out_hbm, sp_table, idx_v, rows_v, out_v, sems):
        cid = lax.axis_index("c")
        sid = lax.axis_index("s")
        wid = sid * 2 + cid
        base = wid * NODES_PER_W

        pltpu.sync_copy(t_hbm.at[pl.ds(sid * STAGE, STAGE)],
                        sp_table.at[pl.ds(sid * STAGE, STAGE)])
        pltpu.sync_copy(adj_hbm.at[wid], idx_v)
        plsc.subcore_barrier()

        def start(chunk, buf):
            pltpu.async_copy(sp_table.at[idx_v.at[chunk]],
                             rows_v[buf], sems[buf])

        for p in range(NBUF - 1):
            start(p, p)

        @pl.loop(0, NCHUNKS, step=NBUF)
        def chunk_quad(g):
            for b in range(NBUF):
                gg = g + b
                nxt = gg + (NBUF - 1)

                @pl.when(nxt < NCHUNKS)
                def _():
                    start(nxt, (b + NBUF - 1) % NBUF)

                pltpu.make_async_copy(
                    sp_table.at[idx_v.at[0]], rows_v[b], sems[b]).wait()
                for n in range(CHUNK):
                    for f in range(NSUB):
                        acc = rows_v[b][n * DEG, pl.ds(16 * f, 16)]
                        for j in range(1, DEG):
                            acc = acc + rows_v[b][n * DEG + j,
                                                  pl.ds(16 * f, 16)]
                        out_v[gg * CHUNK + n, pl.ds(16 * f, 16)] = (
                            acc * (1.0 / DEG))

        pltpu.sync_copy(out_v, out_hbm.at[pl.ds(base, NODES_PER_W)])

    return pl.kernel(
        body,
        out_type=jax.ShapeDtypeStruct((NPAD, KDIM), jnp.float32),
        mesh=plsc.VectorSubcoreMesh(core_axis_name="c", subcore_axis_name="s"),
        scratch_types=[
            pltpu.VMEM_SHARED((table_rows, KDIM), jnp.float32),
            pltpu.VMEM((NCHUNKS, IDXS), jnp.int32),
            [pltpu.VMEM((IDXS, KDIM), jnp.float32) for _ in range(NBUF)],
            pltpu.VMEM((NODES_PER_W, KDIM), jnp.float32),
            [pltpu.SemaphoreType.DMA for _ in range(NBUF)],
        ],
        compiler_params=pltpu.CompilerParams(use_tc_tiling_on_sc=False),
    )


_hop_n = _make_hop(N)
_hop_npad = _make_hop(NPAD)


def _ew1_body(d_ref, o_ref):
    e = jnp.exp(d_ref[...])
    s = jnp.sum(e, axis=1, keepdims=True)
    o_ref[...] = e / jnp.where(s == 0.0, 1.0, s)


_EW_BLOCK = 1000


def _ew1(d):
    return pl.pallas_call(
        _ew1_body,
        grid=(N // _EW_BLOCK,),
        in_specs=[pl.BlockSpec((_EW_BLOCK, KDIM), lambda i: (i, 0))],
        out_specs=pl.BlockSpec((_EW_BLOCK, KDIM), lambda i: (i, 0)),
        out_shape=jax.ShapeDtypeStruct((N, KDIM), jnp.float32),
    )(d)


def _ew2_body(d_ref, o_ref):
    o_ref[...] = jnp.log(jnp.clip(d_ref[...], 1e-9, None))


def _ew2(d):
    return pl.pallas_call(
        _ew2_body,
        grid=(N // _EW_BLOCK,),
        in_specs=[pl.BlockSpec((_EW_BLOCK, KDIM), lambda i: (i, 0))],
        out_specs=pl.BlockSpec((_EW_BLOCK, KDIM), lambda i: (i, 0)),
        out_shape=jax.ShapeDtypeStruct((N, KDIM), jnp.float32),
    )(d)


def _fin_body(q1_ref, q2_ref, b_ref, o_ref):
    o_ref[...] = jnp.dot(
        q1_ref[...] + q2_ref[...],
        b_ref[...],
        precision=lax.Precision.HIGHEST,
        preferred_element_type=jnp.float32,
    )


_FIN_BLOCK = 1000


def _final(q1, q2, b2):
    return pl.pallas_call(
        _fin_body,
        grid=(N // _FIN_BLOCK,),
        in_specs=[
            pl.BlockSpec((_FIN_BLOCK, KDIM), lambda i: (i, 0)),
            pl.BlockSpec((_FIN_BLOCK, KDIM), lambda i: (i, 0)),
            pl.BlockSpec((KDIM, 128), lambda i: (0, 0)),
        ],
        out_specs=pl.BlockSpec((_FIN_BLOCK, 128), lambda i: (i, 0)),
        out_shape=jax.ShapeDtypeStruct((N, 128), jnp.float32),
    )(q1, q2, b2)


def kernel(x, adj_index):
    adj = adj_index.astype(jnp.int32)
    adj = jnp.pad(adj, ((0, NPAD - N), (0, 0)))
    adj_flat = adj.reshape(NW, NCHUNKS, IDXS)

    U, S, Vt = jnp.linalg.svd(x, full_matrices=False)
    dis = U[:, :N_COMPONENT]
    s64 = S[:N_COMPONENT]
    base = Vt[:N_COMPONENT, :]

    qs = []
    for _ in range(LAYER_L):
        t = _ew1(dis)
        h1 = _hop_n(adj_flat, t)
        h2 = _hop_npad(adj_flat, h1)
        a = _ew2(h2)
        q, _ = jnp.linalg.qr(a)
        dis = q
        qs.append(q)

    b2 = (0.5 * s64)[:, None] * base
    return _final(qs[0], qs[1], b2)
